# Initial kernel scaffold; baseline (speedup 1.0000x reference)
import jax
import jax.numpy as jnp
from jax.experimental import pallas as pl


def kernel(indices, embedding_matrix, W, b):
    raise NotImplementedError("write your pallas kernel here")


# speedup vs baseline: 9.8420x; 9.8420x over previous
import functools

import jax
import jax.numpy as jnp
from jax import lax
from jax.experimental import pallas as pl
from jax.experimental.pallas import tpu as pltpu
from jax.experimental.pallas import tpu_sc as plsc

_B = 1024
_S = 520
_EMB = 128
_NC, _NS = 2, 16
_NW = _NC * _NS
_BPW = _B // _NW
_C = 104
_CPE = _S // _C
_NCHUNKS = _BPW * _CPE
_GR = 8
_NG = _C // _GR
_NV = _EMB // 16


def _sc_featurize(idx_flat, table):
    mesh = plsc.VectorSubcoreMesh(
        core_axis_name="c", subcore_axis_name="s",
        num_cores=_NC, num_subcores=_NS)

    @functools.partial(
        pl.kernel,
        out_type=jax.ShapeDtypeStruct((_B, 2 * _EMB), jnp.float32),
        mesh=mesh,
        scratch_types=[
            pltpu.VMEM((_BPW * _S,), jnp.int32),
            pltpu.VMEM((2, _C, _EMB), jnp.float32),
            pltpu.VMEM((_BPW, 2 * _EMB), jnp.float32),
            pltpu.SemaphoreType.DMA,
            pltpu.SemaphoreType.DMA,
        ],
    )
    def k(idx_hbm, table_hbm, feat_hbm, idx_v, rows_v, feat_v, sem0, sem1):
        wid = lax.axis_index("s") * _NC + lax.axis_index("c")
        base_e = wid * _BPW

        dn = lax.GatherDimensionNumbers(
            offset_dims=(), collapsed_slice_dims=(0,), start_index_map=(0,))
        bfly_idx = [(lax.iota(jnp.int32, 16) ^ s).reshape(16, 1)
                    for s in (1, 2, 4, 8)]

        def lanesum(v):
            for idx in bfly_idx:
                v = v + lax.gather(
                    v, idx, dn, slice_sizes=(1,),
                    mode=lax.GatherScatterMode.PROMISE_IN_BOUNDS)
            return v

        pltpu.sync_copy(idx_hbm.at[pl.ds(base_e * _S, _BPW * _S)], idx_v)

        def dma(kc, buf):
            sem = sem0 if buf == 0 else sem1
            return pltpu.make_async_copy(
                table_hbm.at[idx_v.at[pl.ds(kc * _C, _C)]],
                rows_v.at[buf], sem)

        def process_chunk(buf, carry):
            rv = rows_v.at[buf]

            def grp(g, carry):
                accs, bests, bestn = carry
                for r8 in range(_GR):
                    row = g * _GR + r8
                    regs = [rv[row, pl.ds(16 * j, 16)] for j in range(_NV)]
                    sq = regs[0] * regs[0]
                    for j in range(1, _NV):
                        sq = sq + regs[j] * regs[j]
                    nv = lanesum(sq)
                    m = nv > bestn
                    accs = tuple(a + r for a, r in zip(accs, regs))
                    bests = tuple(jnp.where(m, r, bb)
                                  for bb, r in zip(bests, regs))
                    bestn = jnp.maximum(nv, bestn)
                return accs, bests, bestn

            return lax.fori_loop(0, _NG, grp, carry)

        dma(0, 0).start()

        def body_e2(e2, _):
            for eoff in range(2):
                zero = jnp.zeros((16,), jnp.float32)
                carry = (
                    tuple(zero for _ in range(_NV)),
                    tuple(zero for _ in range(_NV)),
                    jnp.full((16,), -1.0, jnp.float32),
                )
                e = e2 * 2 + eoff
                for c in range(_CPE):
                    kc = e * _CPE + c
                    buf = (eoff * _CPE + c) % 2
                    dma(kc, buf).wait()

                    @pl.when(kc + 1 < _NCHUNKS)
                    def _():
                        dma(kc + 1, 1 - buf).start()

                    carry = process_chunk(buf, carry)

                accs, bests, _ = carry
                for j in range(_NV):
                    feat_v[e, pl.ds(16 * j, 16)] = accs[j] * (1.0 / _S)
                    feat_v[e, pl.ds(_EMB + 16 * j, 16)] = bests[j]
            return 0

        lax.fori_loop(0, _BPW // 2, body_e2, 0)
        pltpu.sync_copy(feat_v, feat_hbm.at[pl.ds(base_e, _BPW)])

    return k(idx_flat, table)


def _tc_head(feat, w_pad, b_pad):
    def body(x_ref, w_ref, b_ref, o_ref):
        o_ref[...] = jnp.dot(
            x_ref[...], w_ref[...],
            preferred_element_type=jnp.float32) + b_ref[...]

    return pl.pallas_call(
        body,
        out_shape=jax.ShapeDtypeStruct((_B, 128), jnp.float32),
    )(feat, w_pad, b_pad)


def kernel(indices, embedding_matrix, W, b):
    idx_flat = indices.reshape(-1)
    feat = _sc_featurize(idx_flat, embedding_matrix)
    nclass = W.shape[1]
    w_pad = jnp.zeros((2 * _EMB, 128), jnp.float32).at[:, :nclass].set(W)
    b_pad = jnp.zeros((1, 128), jnp.float32).at[0, :nclass].set(b)
    out = _tc_head(feat, w_pad, b_pad)
    return out[:, :nclass]


# speedup vs baseline: 11.6839x; 1.1871x over previous
import functools

import jax
import jax.numpy as jnp
from jax import lax
from jax.experimental import pallas as pl
from jax.experimental.pallas import tpu as pltpu
from jax.experimental.pallas import tpu_sc as plsc

_B = 1024
_S = 520
_EMB = 128
_NC, _NS = 2, 16
_NW = _NC * _NS
_BPW = _B // _NW
_C = 104
_CPE = _S // _C
_NCHUNKS = _BPW * _CPE
_GR = 8
_NG = _C // _GR
_NV = _EMB // 16
_RING = 4


def _sc_featurize(idx_flat, table):
    mesh = plsc.VectorSubcoreMesh(
        core_axis_name="c", subcore_axis_name="s",
        num_cores=_NC, num_subcores=_NS)

    @functools.partial(
        pl.kernel,
        out_type=jax.ShapeDtypeStruct((_B, 2 * _EMB), jnp.float32),
        mesh=mesh,
        scratch_types=[
            pltpu.VMEM((_BPW * _S,), jnp.int32),
            pltpu.VMEM((_RING, _C, _EMB), jnp.float32),
            pltpu.VMEM((_BPW, 2 * _EMB), jnp.float32),
            pltpu.SemaphoreType.DMA,
            pltpu.SemaphoreType.DMA,
            pltpu.SemaphoreType.DMA,
            pltpu.SemaphoreType.DMA,
        ],
    )
    def k(idx_hbm, table_hbm, feat_hbm, idx_v, rows_v, feat_v,
          sem0, sem1, sem2, sem3):
        wid = lax.axis_index("s") * _NC + lax.axis_index("c")
        base_e = wid * _BPW
        sems = [sem0, sem1, sem2, sem3]

        dn = lax.GatherDimensionNumbers(
            offset_dims=(), collapsed_slice_dims=(0,), start_index_map=(0,))
        bfly_idx = [(lax.iota(jnp.int32, 16) ^ s).reshape(16, 1)
                    for s in (1, 2, 4, 8)]

        def lanesum(v):
            for idx in bfly_idx:
                v = v + lax.gather(
                    v, idx, dn, slice_sizes=(1,),
                    mode=lax.GatherScatterMode.PROMISE_IN_BOUNDS)
            return v

        pltpu.sync_copy(idx_hbm.at[pl.ds(base_e * _S, _BPW * _S)], idx_v)

        def dma(kc, buf):
            return pltpu.make_async_copy(
                table_hbm.at[idx_v.at[pl.ds(kc * _C, _C)]],
                rows_v.at[buf], sems[buf])

        for p in range(_RING - 1):
            dma(p, p).start()

        def body_k(kc, carry):
            accs, bests, bestn = carry
            par = lax.rem(kc, _RING)
            for b in range(_RING):
                @pl.when(par == b)
                def _():
                    dma(kc, b).wait()

                @pl.when(jnp.logical_and(par == b,
                                         kc + _RING - 1 < _NCHUNKS))
                def _():
                    dma(kc + _RING - 1, (b + _RING - 1) % _RING).start()

            first = lax.rem(kc, _CPE) == 0
            keep = jnp.where(first, 0.0, 1.0)
            accs = tuple(a * keep for a in accs)
            bestn = bestn * keep - (1.0 - keep)

            rv = rows_v.at[par]

            def grp(g, carry):
                accs, bests, bestn = carry
                for r8 in range(_GR):
                    row = g * _GR + r8
                    regs = [rv[row, pl.ds(16 * j, 16)] for j in range(_NV)]
                    sq = regs[0] * regs[0]
                    for j in range(1, _NV):
                        sq = sq + regs[j] * regs[j]
                    nv = lanesum(sq)
                    m = nv > bestn
                    accs = tuple(a + r for a, r in zip(accs, regs))
                    bests = tuple(jnp.where(m, r, bb)
                                  for bb, r in zip(bests, regs))
                    bestn = jnp.maximum(nv, bestn)
                return accs, bests, bestn

            accs, bests, bestn = lax.fori_loop(
                0, _NG, grp, (accs, bests, bestn))

            @pl.when(lax.rem(kc, _CPE) == _CPE - 1)
            def _():
                e = kc // _CPE
                for j in range(_NV):
                    feat_v[e, pl.ds(16 * j, 16)] = accs[j] * (1.0 / _S)
                    feat_v[e, pl.ds(_EMB + 16 * j, 16)] = bests[j]
            return accs, bests, bestn

        zero = jnp.zeros((16,), jnp.float32)
        init = (tuple(zero for _ in range(_NV)),
                tuple(zero for _ in range(_NV)),
                jnp.full((16,), -1.0, jnp.float32))
        lax.fori_loop(0, _NCHUNKS, body_k, init)
        pltpu.sync_copy(feat_v, feat_hbm.at[pl.ds(base_e, _BPW)])

    return k(idx_flat, table)


def _tc_head(feat, w_pad, b_pad):
    def body(x_ref, w_ref, b_ref, o_ref):
        o_ref[...] = jnp.dot(
            x_ref[...], w_ref[...],
            preferred_element_type=jnp.float32) + b_ref[...]

    return pl.pallas_call(
        body,
        out_shape=jax.ShapeDtypeStruct((_B, 128), jnp.float32),
    )(feat, w_pad, b_pad)


def kernel(indices, embedding_matrix, W, b):
    idx_flat = indices.reshape(-1)
    feat = _sc_featurize(idx_flat, embedding_matrix)
    nclass = W.shape[1]
    w_pad = jnp.zeros((2 * _EMB, 128), jnp.float32).at[:, :nclass].set(W)
    b_pad = jnp.zeros((1, 128), jnp.float32).at[0, :nclass].set(b)
    out = _tc_head(feat, w_pad, b_pad)
    return out[:, :nclass]
